# baseline (speedup 1.0000x reference)
import functools

import jax
import jax.numpy as jnp
from jax import lax
from jax.experimental import pallas as pl
from jax.experimental.pallas import tpu as pltpu
from jax.experimental.pallas import tpu_sc as plsc

_VOCAB = 100000
_EMB = 32
_MSG_LEN = 200
_H_DIM = 1024
_BATCH = 16384

_NW = 32
_ROWS = _BATCH * _MSG_LEN
_ROWS_W = _ROWS // _NW
_CHUNK = 2048
_IDX_W = 128
_IDX_ROWS = _CHUNK // _IDX_W
_N_CHUNK = _ROWS_W // _CHUNK
_DW = _EMB // 2


def _sc_gather(tab_i32, idx2d):
    mesh = plsc.VectorSubcoreMesh(core_axis_name="c", subcore_axis_name="s")

    @functools.partial(
        pl.kernel,
        mesh=mesh,
        compiler_params=pltpu.CompilerParams(use_tc_tiling_on_sc=False),
        out_type=jax.ShapeDtypeStruct((_ROWS, _DW), jnp.int32),
        scratch_types=[
            pltpu.VMEM((_IDX_ROWS, _IDX_W), jnp.int32),
            pltpu.VMEM((_CHUNK, _DW), jnp.int32),
            pltpu.SemaphoreType.DMA,
        ],
    )
    def k(tab_hbm, idx_hbm, out_hbm, idx_v, rows_v, gsem):
        wid = lax.axis_index("s") * 2 + lax.axis_index("c")
        base = wid * _ROWS_W

        def body(ci, carry):
            row0 = pl.multiple_of(base + ci * _CHUNK, _CHUNK)
            irow0 = pl.multiple_of(row0 // _IDX_W, _IDX_ROWS)
            pltpu.sync_copy(idx_hbm.at[pl.ds(irow0, _IDX_ROWS)], idx_v)
            copies = [
                pltpu.async_copy(
                    tab_hbm.at[idx_v.at[j]],
                    rows_v.at[pl.ds(j * _IDX_W, _IDX_W)],
                    gsem,
                )
                for j in range(_IDX_ROWS)
            ]
            for c in copies:
                c.wait()
            pltpu.sync_copy(rows_v, out_hbm.at[pl.ds(row0, _CHUNK)])
            return carry

        lax.fori_loop(0, _N_CHUNK, body, 0)

    return k(tab_i32, idx2d)


def _tc_mlp(flat_bf, w_bf, b2d):
    bsz, k = flat_bf.shape
    h = w_bf.shape[1]
    bm = 256

    def body(f_ref, w_ref, b_ref, o_ref):
        acc = jnp.dot(f_ref[...], w_ref[...], preferred_element_type=jnp.float32)
        o_ref[...] = jnp.maximum(acc + b_ref[...], 0.0)

    return pl.pallas_call(
        body,
        grid=(bsz // bm,),
        in_specs=[
            pl.BlockSpec((bm, k), lambda i: (i, 0)),
            pl.BlockSpec((k, h), lambda i: (0, 0)),
            pl.BlockSpec((1, h), lambda i: (0, 0)),
        ],
        out_specs=pl.BlockSpec((bm, h), lambda i: (i, 0)),
        out_shape=jax.ShapeDtypeStruct((bsz, h), jnp.float32),
    )(flat_bf, w_bf, b2d)


def kernel(x, table, W, b):
    table_bf = table.astype(jnp.bfloat16)
    tab_i32 = lax.bitcast_convert_type(
        table_bf.reshape(_VOCAB, _DW, 2), jnp.int32
    )
    idx2d = x.reshape(_ROWS // _IDX_W, _IDX_W)
    flat_i32 = _sc_gather(tab_i32, idx2d)
    flat_bf = lax.bitcast_convert_type(flat_i32, jnp.bfloat16).reshape(
        _BATCH, _MSG_LEN * _EMB
    )
    return _tc_mlp(flat_bf, W.astype(jnp.bfloat16), b.reshape(1, _H_DIM))


# speedup vs baseline: 29.7863x; 29.7863x over previous
import functools

import jax
import jax.numpy as jnp
from jax import lax
from jax.experimental import pallas as pl
from jax.experimental.pallas import tpu as pltpu
from jax.experimental.pallas import tpu_sc as plsc

_VOCAB = 100000
_EMB = 32
_MSG_LEN = 200
_H_DIM = 1024
_BATCH = 16384

_BM = 256
_NB = _BATCH // _BM
_NC = _MSG_LEN // 4

_NW = 32
_ROWS = _BATCH * _MSG_LEN
_ROWS_W = _ROWS // _NW
_CHUNK = 2048
_IDX_W = 128
_IDX_ROWS = _CHUNK // _IDX_W
_N_CHUNK = _ROWS_W // _CHUNK
_OUT_R = _CHUNK * _EMB // 128


def _sc_gather(table, idx2d):
    mesh = plsc.VectorSubcoreMesh(core_axis_name="c", subcore_axis_name="s")

    @functools.partial(
        pl.kernel,
        mesh=mesh,
        compiler_params=pltpu.CompilerParams(use_tc_tiling_on_sc=False),
        out_type=jax.ShapeDtypeStruct((_ROWS * _EMB // 128, 128), jnp.float32),
        scratch_types=[
            pltpu.VMEM((_IDX_ROWS, _IDX_W), jnp.int32),
            pltpu.VMEM((2, _IDX_W, _EMB), jnp.float32),
            pltpu.VMEM((_OUT_R, 128), jnp.float32),
            pltpu.SemaphoreType.DMA,
            pltpu.SemaphoreType.DMA,
            pltpu.SemaphoreType.DMA,
        ],
    )
    def k(tab_hbm, idx_hbm, out_hbm, idx_v, ga, rows2_v, g0, g1, osem):
        wid = lax.axis_index("s") * 2 + lax.axis_index("c")
        base = wid * _ROWS_W
        gsems = (g0, g1)

        def copy_block(j):
            gb = ga.at[j % 2]

            def copy4(n4, carry):
                n = pl.multiple_of(n4 * 4, 4)
                for dn in range(4):
                    for q in range(4):
                        for s in range(2):
                            v = gb[(n + dn) * 4 + q, pl.ds(16 * s, 16)]
                            rows2_v[32 * j + n + dn, pl.ds(32 * q + 16 * s, 16)] = v
                return carry

            lax.fori_loop(0, 8, copy4, 0)

        def body(ci, carry):
            row0 = pl.multiple_of(base + ci * _CHUNK, _CHUNK)
            irow0 = pl.multiple_of(row0 // _IDX_W, _IDX_ROWS)
            orow0 = pl.multiple_of(row0 * _EMB // 128, _OUT_R)
            pltpu.sync_copy(idx_hbm.at[pl.ds(irow0, _IDX_ROWS)], idx_v)
            gwaits = [None, None]
            for j in range(_IDX_ROWS):
                gwaits[j % 2] = pltpu.async_copy(
                    tab_hbm.at[idx_v.at[j]], ga.at[j % 2], gsems[j % 2]
                )
                if j == 1:
                    @pl.when(ci > 0)
                    def _():
                        pltpu.make_async_copy(rows2_v, out_hbm.at[pl.ds(0, _OUT_R)], osem).wait()
                if j >= 1:
                    gwaits[(j - 1) % 2].wait()
                    copy_block(j - 1)
            gwaits[(_IDX_ROWS - 1) % 2].wait()
            copy_block(_IDX_ROWS - 1)
            pltpu.async_copy(rows2_v, out_hbm.at[pl.ds(orow0, _OUT_R)], osem)
            return carry

        lax.fori_loop(0, _N_CHUNK, body, 0)
        pltpu.make_async_copy(rows2_v, out_hbm.at[pl.ds(0, _OUT_R)], osem).wait()

    return k(table, idx2d)


def _tc_mlp(fl2, w_bf, b2d):
    def body(f_ref, w_ref, b_ref, o_ref):
        acc = jnp.zeros((_BM, _H_DIM), jnp.float32) + b_ref[...]
        for c in range(_NC):
            f_c = f_ref[pl.ds(c * _BM, _BM), :].astype(jnp.bfloat16)
            w_c = w_ref[pl.ds(c * 128, 128), :]
            acc = acc + jnp.dot(f_c, w_c, preferred_element_type=jnp.float32)
        o_ref[...] = jnp.maximum(acc, 0.0)

    return pl.pallas_call(
        body,
        grid=(_NB,),
        in_specs=[
            pl.BlockSpec((_NC * _BM, 128), lambda i: (i, 0)),
            pl.BlockSpec((_MSG_LEN * _EMB, _H_DIM), lambda i: (0, 0)),
            pl.BlockSpec((1, _H_DIM), lambda i: (0, 0)),
        ],
        out_specs=pl.BlockSpec((_BM, _H_DIM), lambda i: (i, 0)),
        out_shape=jax.ShapeDtypeStruct((_BATCH, _H_DIM), jnp.float32),
    )(fl2, w_bf, b2d)


def kernel(x, table, W, b):
    xp = (
        x.reshape(_NB, _BM, _NC, 4)
        .transpose(0, 2, 1, 3)
        .reshape(_ROWS // _IDX_W, _IDX_W)
    )
    fl2 = _sc_gather(table, xp)
    return _tc_mlp(fl2, W.astype(jnp.bfloat16), b.reshape(1, _H_DIM))


# speedup vs baseline: 48.3529x; 1.6233x over previous
import functools

import jax
import jax.numpy as jnp
from jax import lax
from jax.experimental import pallas as pl
from jax.experimental.pallas import tpu as pltpu
from jax.experimental.pallas import tpu_sc as plsc

_VOCAB = 100000
_EMB = 32
_MSG_LEN = 200
_H_DIM = 1024
_BATCH = 16384

_BM = 256
_NB = _BATCH // _BM
_NC = _MSG_LEN // 4

_NW = 32
_ROWS = _BATCH * _MSG_LEN
_ROWS_W = _ROWS // _NW
_CHUNK = 2048
_IDX_W = 128
_IDX_ROWS = _CHUNK // _IDX_W
_N_CHUNK = _ROWS_W // _CHUNK
_OUT_R = _CHUNK * _EMB // 128


def _sc_gather(table, idx2d):
    mesh = plsc.VectorSubcoreMesh(core_axis_name="c", subcore_axis_name="s")

    @functools.partial(
        pl.kernel,
        mesh=mesh,
        compiler_params=pltpu.CompilerParams(use_tc_tiling_on_sc=False),
        out_type=jax.ShapeDtypeStruct((_ROWS * _EMB // 128, 128), jnp.float32),
        scratch_types=[
            pltpu.VMEM((_IDX_ROWS, _IDX_W), jnp.int32),
            pltpu.VMEM((2, _IDX_W, _EMB), jnp.float32),
            pltpu.VMEM((_OUT_R, 128), jnp.float32),
            pltpu.SemaphoreType.DMA,
            pltpu.SemaphoreType.DMA,
            pltpu.SemaphoreType.DMA,
        ],
    )
    def k(tab_hbm, idx_hbm, out_hbm, idx_v, ga, rows2_v, g0, g1, osem):
        wid = lax.axis_index("s") * 2 + lax.axis_index("c")
        base = wid * _ROWS_W
        gsems = (g0, g1)

        def copy_block(j):
            gb = ga.at[j % 2]

            def copy4(n4, carry):
                n = pl.multiple_of(n4 * 4, 4)
                for dn in range(4):
                    for q in range(4):
                        for s in range(2):
                            v = gb[(n + dn) * 4 + q, pl.ds(16 * s, 16)]
                            rows2_v[32 * j + n + dn, pl.ds(32 * q + 16 * s, 16)] = v
                return carry

            lax.fori_loop(0, 8, copy4, 0)

        def body(ci, carry):
            row0 = pl.multiple_of(base + ci * _CHUNK, _CHUNK)
            irow0 = pl.multiple_of(row0 // _IDX_W, _IDX_ROWS)
            orow0 = pl.multiple_of(row0 * _EMB // 128, _OUT_R)
            pltpu.sync_copy(idx_hbm.at[pl.ds(irow0, _IDX_ROWS)], idx_v)
            gwaits = [None, None]
            for j in range(_IDX_ROWS):
                gwaits[j % 2] = pltpu.async_copy(
                    tab_hbm.at[idx_v.at[j]], ga.at[j % 2], gsems[j % 2]
                )
                if j == 1:
                    @pl.when(ci > 0)
                    def _():
                        pltpu.make_async_copy(rows2_v, out_hbm.at[pl.ds(0, _OUT_R)], osem).wait()
                if j >= 1:
                    gwaits[(j - 1) % 2].wait()
                    copy_block(j - 1)
            gwaits[(_IDX_ROWS - 1) % 2].wait()
            copy_block(_IDX_ROWS - 1)
            pltpu.async_copy(rows2_v, out_hbm.at[pl.ds(orow0, _OUT_R)], osem)
            return carry

        lax.fori_loop(0, _N_CHUNK, body, 0)
        pltpu.make_async_copy(rows2_v, out_hbm.at[pl.ds(0, _OUT_R)], osem).wait()

    return k(table, idx2d)


def _tc_mlp(fl2, w_bf, b2d):
    def body(f_ref, w_ref, b_ref, o_ref):
        acc = jnp.zeros((_BM, _H_DIM), jnp.float32) + b_ref[...]
        for c in range(_NC):
            f_c = f_ref[pl.ds(c * _BM, _BM), :].astype(jnp.bfloat16)
            w_c = w_ref[pl.ds(c * 128, 128), :]
            acc = acc + jnp.dot(f_c, w_c, preferred_element_type=jnp.float32)
        o_ref[...] = jnp.maximum(acc, 0.0)

    return pl.pallas_call(
        body,
        grid=(_NB,),
        in_specs=[
            pl.BlockSpec((_NC * _BM, 128), lambda i: (i, 0)),
            pl.BlockSpec((_MSG_LEN * _EMB, _H_DIM), lambda i: (0, 0)),
            pl.BlockSpec((1, _H_DIM), lambda i: (0, 0)),
        ],
        out_specs=pl.BlockSpec((_BM, _H_DIM), lambda i: (i, 0)),
        out_shape=jax.ShapeDtypeStruct((_BATCH, _H_DIM), jnp.float32),
    )(fl2, w_bf, b2d)


def _tc_permute(x):
    def body(x_ref, o_ref):
        xb = x_ref[...]
        o_ref[...] = (
            xb.reshape(8, 32, _NC, 4).transpose(2, 0, 1, 3).reshape(8 * _NC, 128)
        )

    return pl.pallas_call(
        body,
        grid=(_NB,),
        in_specs=[pl.BlockSpec((_BM, _MSG_LEN), lambda i: (i, 0))],
        out_specs=pl.BlockSpec((8 * _NC, 128), lambda i: (i, 0)),
        out_shape=jax.ShapeDtypeStruct((_ROWS // _IDX_W, _IDX_W), jnp.int32),
    )(x)


def kernel(x, table, W, b):
    xp = _tc_permute(x)
    fl2 = _sc_gather(table, xp)
    return _tc_mlp(fl2, W.astype(jnp.bfloat16), b.reshape(1, _H_DIM))


# speedup vs baseline: 58.5559x; 1.2110x over previous
import functools

import jax
import jax.numpy as jnp
from jax import lax
from jax.experimental import pallas as pl
from jax.experimental.pallas import tpu as pltpu
from jax.experimental.pallas import tpu_sc as plsc

_VOCAB = 100000
_EMB = 32
_MSG_LEN = 200
_H_DIM = 1024
_BATCH = 16384

_BM = 512
_NC = _MSG_LEN // 4

_NW = 32
_CHUNK = 2048
_IDX_W = 128
_IDX_ROWS = _CHUNK // _IDX_W
_OUT_R = _CHUNK * _EMB // 128

_NSLICE = 2


def _sc_gather(table, idx2d):
    rows = idx2d.shape[0] * _IDX_W
    rows_w = rows // _NW
    n_chunk = rows_w // _CHUNK
    mesh = plsc.VectorSubcoreMesh(core_axis_name="c", subcore_axis_name="s")

    @functools.partial(
        pl.kernel,
        mesh=mesh,
        compiler_params=pltpu.CompilerParams(use_tc_tiling_on_sc=False),
        out_type=jax.ShapeDtypeStruct((rows * _EMB // 128, 128), jnp.float32),
        scratch_types=[
            pltpu.VMEM((_IDX_ROWS, _IDX_W), jnp.int32),
            pltpu.VMEM((2, _IDX_W, _EMB), jnp.float32),
            pltpu.VMEM((_OUT_R, 128), jnp.float32),
            pltpu.SemaphoreType.DMA,
            pltpu.SemaphoreType.DMA,
            pltpu.SemaphoreType.DMA,
        ],
    )
    def k(tab_hbm, idx_hbm, out_hbm, idx_v, ga, rows2_v, g0, g1, osem):
        wid = lax.axis_index("s") * 2 + lax.axis_index("c")
        base = wid * rows_w
        gsems = (g0, g1)

        def copy_block(j):
            gb = ga.at[j % 2]

            def copy4(n4, carry):
                n = pl.multiple_of(n4 * 4, 4)
                for dn in range(4):
                    for q in range(4):
                        for s in range(2):
                            v = gb[(n + dn) * 4 + q, pl.ds(16 * s, 16)]
                            rows2_v[32 * j + n + dn, pl.ds(32 * q + 16 * s, 16)] = v
                return carry

            lax.fori_loop(0, 8, copy4, 0)

        def body(ci, carry):
            row0 = pl.multiple_of(base + ci * _CHUNK, _CHUNK)
            irow0 = pl.multiple_of(row0 // _IDX_W, _IDX_ROWS)
            orow0 = pl.multiple_of(row0 * _EMB // 128, _OUT_R)
            pltpu.sync_copy(idx_hbm.at[pl.ds(irow0, _IDX_ROWS)], idx_v)
            gwaits = [None, None]
            for j in range(_IDX_ROWS):
                gwaits[j % 2] = pltpu.async_copy(
                    tab_hbm.at[idx_v.at[j]], ga.at[j % 2], gsems[j % 2]
                )
                if j == 1:
                    @pl.when(ci > 0)
                    def _():
                        pltpu.make_async_copy(
                            rows2_v, out_hbm.at[pl.ds(0, _OUT_R)], osem
                        ).wait()
                if j >= 1:
                    gwaits[(j - 1) % 2].wait()
                    copy_block(j - 1)
            gwaits[(_IDX_ROWS - 1) % 2].wait()
            copy_block(_IDX_ROWS - 1)
            pltpu.async_copy(rows2_v, out_hbm.at[pl.ds(orow0, _OUT_R)], osem)
            return carry

        lax.fori_loop(0, n_chunk, body, 0)
        pltpu.make_async_copy(rows2_v, out_hbm.at[pl.ds(0, _OUT_R)], osem).wait()

    return k(table, idx2d)


def _tc_mlp(fl2, w_bf, b2d):
    nb = fl2.shape[0] // (_NC * _BM)
    bsz = nb * _BM

    def body(f_ref, w_ref, b_ref, o_ref):
        accs = [jnp.zeros((_BM, _H_DIM), jnp.float32) for _ in range(4)]
        for c in range(_NC):
            f_c = f_ref[pl.ds(c * _BM, _BM), :].astype(jnp.bfloat16)
            w_c = w_ref[pl.ds(c * 128, 128), :]
            accs[c % 4] = accs[c % 4] + jnp.dot(
                f_c, w_c, preferred_element_type=jnp.float32
            )
        acc = (accs[0] + accs[1]) + (accs[2] + accs[3]) + b_ref[...]
        o_ref[...] = jnp.maximum(acc, 0.0)

    return pl.pallas_call(
        body,
        grid=(nb,),
        in_specs=[
            pl.BlockSpec((_NC * _BM, 128), lambda i: (i, 0)),
            pl.BlockSpec((_MSG_LEN * _EMB, _H_DIM), lambda i: (0, 0)),
            pl.BlockSpec((1, _H_DIM), lambda i: (0, 0)),
        ],
        out_specs=pl.BlockSpec((_BM, _H_DIM), lambda i: (i, 0)),
        out_shape=jax.ShapeDtypeStruct((bsz, _H_DIM), jnp.float32),
    )(fl2, w_bf, b2d)


def _tc_permute(x):
    bsz = x.shape[0]
    nb = bsz // _BM

    def body(x_ref, o_ref):
        xb = x_ref[...]
        o_ref[...] = (
            xb.reshape(_BM // 32, 32, _NC, 4)
            .transpose(2, 0, 1, 3)
            .reshape(_BM * _NC // 32, 128)
        )

    return pl.pallas_call(
        body,
        grid=(nb,),
        in_specs=[pl.BlockSpec((_BM, _MSG_LEN), lambda i: (i, 0))],
        out_specs=pl.BlockSpec((_BM * _NC // 32, 128), lambda i: (i, 0)),
        out_shape=jax.ShapeDtypeStruct((bsz * _MSG_LEN // _IDX_W, _IDX_W), jnp.int32),
    )(x)


def kernel(x, table, W, b):
    w_bf = W.astype(jnp.bfloat16)
    b2d = b.reshape(1, _H_DIM)
    bs = _BATCH // _NSLICE
    outs = []
    for s in range(_NSLICE):
        xs = lax.slice_in_dim(x, s * bs, (s + 1) * bs, axis=0)
        xp = _tc_permute(xs)
        fl2 = _sc_gather(table, xp)
        outs.append(_tc_mlp(fl2, w_bf, b2d))
    return lax.concatenate(outs, 0)


# speedup vs baseline: 66.2717x; 1.1318x over previous
import functools

import jax
import jax.numpy as jnp
from jax import lax
from jax.experimental import pallas as pl
from jax.experimental.pallas import tpu as pltpu
from jax.experimental.pallas import tpu_sc as plsc

_VOCAB = 100000
_EMB = 32
_MSG_LEN = 200
_H_DIM = 1024
_BATCH = 16384

_BM = 512
_NC = _MSG_LEN // 4

_NW = 32
_CHUNK = 1024
_IDX_W = 128
_IDX_ROWS = _CHUNK // _IDX_W
_OUT_R = _CHUNK * _EMB // 128

_NSLICE = 4


def _sc_gather(table, idx2d):
    rows = idx2d.shape[0] * _IDX_W
    rows_w = rows // _NW
    n_chunk = rows_w // _CHUNK
    mesh = plsc.VectorSubcoreMesh(core_axis_name="c", subcore_axis_name="s")

    @functools.partial(
        pl.kernel,
        mesh=mesh,
        compiler_params=pltpu.CompilerParams(use_tc_tiling_on_sc=False),
        out_type=jax.ShapeDtypeStruct((rows * _EMB // 128, 128), jnp.float32),
        scratch_types=[
            pltpu.VMEM((_IDX_ROWS, _IDX_W), jnp.int32),
            pltpu.VMEM((2, _IDX_W, _EMB), jnp.float32),
            pltpu.VMEM((_OUT_R, 128), jnp.float32),
            pltpu.SemaphoreType.DMA,
            pltpu.SemaphoreType.DMA,
            pltpu.SemaphoreType.DMA,
        ],
    )
    def k(tab_hbm, idx_hbm, out_hbm, idx_v, ga, rows2_v, g0, g1, osem):
        wid = lax.axis_index("s") * 2 + lax.axis_index("c")
        base = wid * rows_w
        gsems = (g0, g1)

        def copy_block(j):
            gb = ga.at[j % 2]

            def copy4(n4, carry):
                n = pl.multiple_of(n4 * 4, 4)
                for dn in range(4):
                    for q in range(4):
                        for s in range(2):
                            v = gb[(n + dn) * 4 + q, pl.ds(16 * s, 16)]
                            rows2_v[32 * j + n + dn, pl.ds(32 * q + 16 * s, 16)] = v
                return carry

            lax.fori_loop(0, 8, copy4, 0)

        def body(ci, carry):
            row0 = pl.multiple_of(base + ci * _CHUNK, _CHUNK)
            irow0 = pl.multiple_of(row0 // _IDX_W, _IDX_ROWS)
            orow0 = pl.multiple_of(row0 * _EMB // 128, _OUT_R)
            pltpu.sync_copy(idx_hbm.at[pl.ds(irow0, _IDX_ROWS)], idx_v)
            gwaits = [None, None]
            for j in range(_IDX_ROWS):
                gwaits[j % 2] = pltpu.async_copy(
                    tab_hbm.at[idx_v.at[j]], ga.at[j % 2], gsems[j % 2]
                )
                if j == 1:
                    @pl.when(ci > 0)
                    def _():
                        pltpu.make_async_copy(
                            rows2_v, out_hbm.at[pl.ds(0, _OUT_R)], osem
                        ).wait()
                if j >= 1:
                    gwaits[(j - 1) % 2].wait()
                    copy_block(j - 1)
            gwaits[(_IDX_ROWS - 1) % 2].wait()
            copy_block(_IDX_ROWS - 1)
            pltpu.async_copy(rows2_v, out_hbm.at[pl.ds(orow0, _OUT_R)], osem)
            return carry

        lax.fori_loop(0, n_chunk, body, 0)
        pltpu.make_async_copy(rows2_v, out_hbm.at[pl.ds(0, _OUT_R)], osem).wait()

    return k(table, idx2d)


def _tc_mlp(fl2, w_bf, b2d):
    nb = fl2.shape[0] // (_NC * _BM)
    bsz = nb * _BM

    def body(f_ref, w_ref, b_ref, o_ref):
        accs = [jnp.zeros((_BM, _H_DIM), jnp.float32) for _ in range(4)]
        for c in range(_NC):
            f_c = f_ref[pl.ds(c * _BM, _BM), :].astype(jnp.bfloat16)
            w_c = w_ref[pl.ds(c * 128, 128), :]
            accs[c % 4] = accs[c % 4] + jnp.dot(
                f_c, w_c, preferred_element_type=jnp.float32
            )
        acc = (accs[0] + accs[1]) + (accs[2] + accs[3]) + b_ref[...]
        o_ref[...] = jnp.maximum(acc, 0.0)

    return pl.pallas_call(
        body,
        grid=(nb,),
        in_specs=[
            pl.BlockSpec((_NC * _BM, 128), lambda i: (i, 0)),
            pl.BlockSpec((_MSG_LEN * _EMB, _H_DIM), lambda i: (0, 0)),
            pl.BlockSpec((1, _H_DIM), lambda i: (0, 0)),
        ],
        out_specs=pl.BlockSpec((_BM, _H_DIM), lambda i: (i, 0)),
        out_shape=jax.ShapeDtypeStruct((bsz, _H_DIM), jnp.float32),
    )(fl2, w_bf, b2d)


def _tc_permute(x):
    bsz = x.shape[0]
    nb = bsz // _BM

    def body(x_ref, o_ref):
        xb = x_ref[...]
        o_ref[...] = (
            xb.reshape(_BM // 32, 32, _NC, 4)
            .transpose(2, 0, 1, 3)
            .reshape(_BM * _NC // 32, 128)
        )

    return pl.pallas_call(
        body,
        grid=(nb,),
        in_specs=[pl.BlockSpec((_BM, _MSG_LEN), lambda i: (i, 0))],
        out_specs=pl.BlockSpec((_BM * _NC // 32, 128), lambda i: (i, 0)),
        out_shape=jax.ShapeDtypeStruct((bsz * _MSG_LEN // _IDX_W, _IDX_W), jnp.int32),
    )(x)


def kernel(x, table, W, b):
    w_bf = W.astype(jnp.bfloat16)
    b2d = b.reshape(1, _H_DIM)
    bs = _BATCH // _NSLICE
    outs = []
    for s in range(_NSLICE):
        xs = lax.slice_in_dim(x, s * bs, (s + 1) * bs, axis=0)
        xp = _tc_permute(xs)
        fl2 = _sc_gather(table, xp)
        outs.append(_tc_mlp(fl2, w_bf, b2d))
    return lax.concatenate(outs, 0)
